# baseline (speedup 1.0000x reference)
import jax
import jax.numpy as jnp
from jax import lax
from jax.experimental import pallas as pl
from jax.experimental.pallas import tpu as pltpu
from jax.experimental.pallas import tpu_sc as plsc

N_USERS = 20000
N_ITEMS = 30000
N_NODES = N_USERS + N_ITEMS
N_EDGES = 800000
EMBED_DIM = 64
N_LAYERS = 3

NC = 2
NS = 16
L = 16

NPAD = 51200
DH = EMBED_DIM // NC

ZR = NPAD // NS
NQ = 16
QR = ZR // NQ

E_TILE = N_EDGES // NS
K = 80
G = 25
BLK = K * G
NB = E_TILE // BLK


def _layer_body(p_prev, acc_prev, rows3, cols3, vals3,
                p_next, acc_next,
                table, zbuf, abuf, gbufa, gbufb, colv2, rowv2, valv2,
                gsa, gsb, ssa, ssb):
    c = lax.axis_index("c")
    s = lax.axis_index("s")

    gbufs = (gbufa, gbufb)
    gsems = (gsa, gsb)
    ssems = (ssa, ssb)

    def _zfill(r, carry):
        for j in range(DH // L):
            zbuf[r, pl.ds(j * L, L)] = jnp.zeros((L,), jnp.float32)
        return carry
    lax.fori_loop(0, QR, _zfill, 0)
    for q in range(NQ):
        pltpu.sync_copy(zbuf, table.at[pl.ds(s * ZR + q * QR, QR)])
    plsc.subcore_barrier()

    psrc = p_prev.at[c]

    def _mul(g, buf):
        @plsc.parallel_loop(0, K // L)
        def _mq(q):
            vv16 = valv2[g, pl.ds(q * L, L)]
            for i in range(L):
                vv = jnp.full((L,), vv16[i], jnp.float32)
                for j in range(DH // L):
                    sl = (q * L + i, pl.ds(j * L, L))
                    buf[sl] = buf[sl] * vv

    def _block(b, carry):
        pltpu.sync_copy(rows3.at[s * NB + b], rowv2)
        pltpu.sync_copy(cols3.at[s * NB + b], colv2)
        pltpu.sync_copy(vals3.at[s * NB + b], valv2)

        gd = [None] * G
        sd = [None] * G
        gd[0] = pltpu.async_copy(psrc.at[colv2.at[0]], gbufs[0], gsems[0])
        for g in range(1, G):
            x = g % 2
            gd[g] = pltpu.async_copy(psrc.at[colv2.at[g]], gbufs[x], gsems[x])
            y = (g - 1) % 2
            gd[g - 1].wait()
            _mul(g - 1, gbufs[y])
        y = (G - 1) % 2
        gd[G - 1].wait()
        _mul(G - 1, gbufs[y])
        return carry
    lax.fori_loop(0, NB, _block, 0)
    plsc.subcore_barrier()

    for q in range(NQ):
        tro = s * ZR + q * QR
        pltpu.sync_copy(table.at[pl.ds(tro, QR)], zbuf)
        pltpu.sync_copy(zbuf, p_next.at[c].at[pl.ds(tro, QR)])
        pltpu.sync_copy(acc_prev.at[c].at[pl.ds(tro, QR)], abuf)

        def _acc(r, carry):
            for j in range(DH // L):
                abuf[r, pl.ds(j * L, L)] = (abuf[r, pl.ds(j * L, L)]
                                            + zbuf[r, pl.ds(j * L, L)])
            return carry
        lax.fori_loop(0, QR, _acc, 0)
        pltpu.sync_copy(abuf, acc_next.at[c].at[pl.ds(tro, QR)])


_layer = pl.kernel(
    _layer_body,
    out_type=(
        jax.ShapeDtypeStruct((NC, NPAD, DH), jnp.float32),
        jax.ShapeDtypeStruct((NC, NPAD, DH), jnp.float32),
    ),
    mesh=plsc.VectorSubcoreMesh(core_axis_name="c", subcore_axis_name="s",
                                num_cores=NC, num_subcores=NS),
    compiler_params=pltpu.CompilerParams(use_tc_tiling_on_sc=False),
    scratch_types=[
        pltpu.VMEM_SHARED((NPAD, DH), jnp.float32),
        pltpu.VMEM((QR, DH), jnp.float32),
        pltpu.VMEM((QR, DH), jnp.float32),
        pltpu.VMEM((K, DH), jnp.float32),
        pltpu.VMEM((K, DH), jnp.float32),
        pltpu.VMEM((G, K), jnp.int32),
        pltpu.VMEM((G, K), jnp.int32),
        pltpu.VMEM((G, K), jnp.float32),
        pltpu.SemaphoreType.DMA,
        pltpu.SemaphoreType.DMA,
        pltpu.SemaphoreType.DMA,
        pltpu.SemaphoreType.DMA,
    ],
)


def kernel(user_preference, item_preference, edge_values, edge_index):
    p0 = jnp.concatenate([user_preference, item_preference], axis=0)
    p0 = jnp.pad(p0, ((0, NPAD - N_NODES), (0, 0)))
    p = jnp.stack([p0[:, :DH], p0[:, DH:]])
    acc = p
    rows3 = edge_index[0].astype(jnp.int32).reshape(NS * NB, G, K)
    cols3 = edge_index[1].astype(jnp.int32).reshape(NS * NB, G, K)
    vals3 = edge_values.astype(jnp.float32).reshape(NS * NB, G, K)
    for _ in range(N_LAYERS):
        p, acc = _layer(p, acc, rows3, cols3, vals3)
    full = jnp.concatenate([acc[0], acc[1]], axis=1)[:N_NODES]
    return (full[:N_USERS], full[N_USERS:])


# speedup vs baseline: 1.0976x; 1.0976x over previous
import jax
import jax.numpy as jnp
from jax import lax
from jax.experimental import pallas as pl
from jax.experimental.pallas import tpu as pltpu
from jax.experimental.pallas import tpu_sc as plsc

N_USERS = 20000
N_ITEMS = 30000
N_NODES = N_USERS + N_ITEMS
N_EDGES = 800000
EMBED_DIM = 64
N_LAYERS = 3

NC = 2
NS = 16
L = 16

NPAD = 51200
DH = EMBED_DIM // NC

ZR = NPAD // NS
NQ = 20
QR = ZR // NQ
NBUF = 4

E_TILE = N_EDGES // NS
K = 80
G = 25
BLK = K * G
NB = E_TILE // BLK


def _layer_body(p_prev, acc_prev, rows3, cols3, vals3,
                p_next, acc_next,
                table, zbuf, abuf,
                gbufa, gbufb, gbufc, gbufd, colv2, rowv2, valv2,
                gsa, gsb, gsc, gsd, ssa, ssb, ssc, ssd):
    c = lax.axis_index("c")
    s = lax.axis_index("s")

    gbufs = (gbufa, gbufb, gbufc, gbufd)
    gsems = (gsa, gsb, gsc, gsd)
    ssems = (ssa, ssb, ssc, ssd)

    def _zfill(r, carry):
        for j in range(DH // L):
            zbuf[r, pl.ds(j * L, L)] = jnp.zeros((L,), jnp.float32)
        return carry
    lax.fori_loop(0, QR, _zfill, 0)
    for q in range(NQ):
        pltpu.sync_copy(zbuf, table.at[pl.ds(s * ZR + q * QR, QR)])
    plsc.subcore_barrier()

    psrc = p_prev.at[c]

    def _mul(g, buf):
        @plsc.parallel_loop(0, K // L)
        def _mq(q):
            vv16 = valv2[g, pl.ds(q * L, L)]
            for i in range(L):
                vv = jnp.full((L,), vv16[i], jnp.float32)
                for j in range(DH // L):
                    sl = (q * L + i, pl.ds(j * L, L))
                    buf[sl] = buf[sl] * vv

    def _block(b, carry):
        pltpu.sync_copy(rows3.at[s * NB + b], rowv2)
        pltpu.sync_copy(cols3.at[s * NB + b], colv2)
        pltpu.sync_copy(vals3.at[s * NB + b], valv2)

        gd = [None] * G
        sd = [None] * G
        for g in range(NBUF - 1):
            x = g % NBUF
            gd[g] = pltpu.async_copy(psrc.at[colv2.at[g]], gbufs[x], gsems[x])
        for gg in range(NBUF - 1, G + NBUF - 1):
            if gg < G:
                x = gg % NBUF
                if gg >= NBUF:
                    sd[gg - NBUF].wait()
                gd[gg] = pltpu.async_copy(psrc.at[colv2.at[gg]], gbufs[x],
                                          gsems[x])
            p = gg - (NBUF - 1)
            y = p % NBUF
            gd[p].wait()
            _mul(p, gbufs[y])
            sd[p] = pltpu.async_copy(gbufs[y], table.at[rowv2.at[p]],
                                     ssems[y], add=True)
        for p in range(G - NBUF, G):
            if p >= 0:
                sd[p].wait()
        return carry
    lax.fori_loop(0, NB, _block, 0)
    plsc.subcore_barrier()

    for q in range(NQ):
        tro = s * ZR + q * QR
        pltpu.sync_copy(table.at[pl.ds(tro, QR)], zbuf)
        pltpu.sync_copy(zbuf, p_next.at[c].at[pl.ds(tro, QR)])
        pltpu.sync_copy(acc_prev.at[c].at[pl.ds(tro, QR)], abuf)

        def _acc(r, carry):
            for j in range(DH // L):
                abuf[r, pl.ds(j * L, L)] = (abuf[r, pl.ds(j * L, L)]
                                            + zbuf[r, pl.ds(j * L, L)])
            return carry
        lax.fori_loop(0, QR, _acc, 0)
        pltpu.sync_copy(abuf, acc_next.at[c].at[pl.ds(tro, QR)])


_layer = pl.kernel(
    _layer_body,
    out_type=(
        jax.ShapeDtypeStruct((NC, NPAD, DH), jnp.float32),
        jax.ShapeDtypeStruct((NC, NPAD, DH), jnp.float32),
    ),
    mesh=plsc.VectorSubcoreMesh(core_axis_name="c", subcore_axis_name="s",
                                num_cores=NC, num_subcores=NS),
    compiler_params=pltpu.CompilerParams(use_tc_tiling_on_sc=False),
    scratch_types=[
        pltpu.VMEM_SHARED((NPAD, DH), jnp.float32),
        pltpu.VMEM((QR, DH), jnp.float32),
        pltpu.VMEM((QR, DH), jnp.float32),
        pltpu.VMEM((K, DH), jnp.float32),
        pltpu.VMEM((K, DH), jnp.float32),
        pltpu.VMEM((K, DH), jnp.float32),
        pltpu.VMEM((K, DH), jnp.float32),
        pltpu.VMEM((G, K), jnp.int32),
        pltpu.VMEM((G, K), jnp.int32),
        pltpu.VMEM((G, K), jnp.float32),
        pltpu.SemaphoreType.DMA,
        pltpu.SemaphoreType.DMA,
        pltpu.SemaphoreType.DMA,
        pltpu.SemaphoreType.DMA,
        pltpu.SemaphoreType.DMA,
        pltpu.SemaphoreType.DMA,
        pltpu.SemaphoreType.DMA,
        pltpu.SemaphoreType.DMA,
    ],
)


def kernel(user_preference, item_preference, edge_values, edge_index):
    p0 = jnp.concatenate([user_preference, item_preference], axis=0)
    p0 = jnp.pad(p0, ((0, NPAD - N_NODES), (0, 0)))
    p = jnp.stack([p0[:, :DH], p0[:, DH:]])
    acc = p
    rows3 = edge_index[0].astype(jnp.int32).reshape(NS * NB, G, K)
    cols3 = edge_index[1].astype(jnp.int32).reshape(NS * NB, G, K)
    vals3 = edge_values.astype(jnp.float32).reshape(NS * NB, G, K)
    for _ in range(N_LAYERS):
        p, acc = _layer(p, acc, rows3, cols3, vals3)
    full = jnp.concatenate([acc[0], acc[1]], axis=1)[:N_NODES]
    return (full[:N_USERS], full[N_USERS:])
